# Initial kernel scaffold; baseline (speedup 1.0000x reference)
import jax
import jax.numpy as jnp
from jax.experimental import pallas as pl


def kernel(x, gate_w, W1, b1, W2, b2):
    raise NotImplementedError("write your pallas kernel here")


# speedup vs baseline: 2.2240x; 2.2240x over previous
import functools

import jax
import jax.numpy as jnp
from jax.experimental import pallas as pl
from jax.experimental.pallas import tpu as pltpu

E = 8
K = 2
D = 1024
F = 4096

BLK_M = 256
BLK_F = 1024


def _mlp_body(gmap, xs_ref, w1_ref, b1_ref, w2_ref, b2_ref, out_ref):
    f = pl.program_id(1)
    x = xs_ref[...]
    h = jnp.dot(x, w1_ref[0], preferred_element_type=jnp.float32) + b1_ref[0]
    h = jax.nn.gelu(h)
    contrib = jnp.dot(h, w2_ref[0], preferred_element_type=jnp.float32)

    @pl.when(f == 0)
    def _():
        out_ref[...] = contrib + b2_ref[0]

    @pl.when(f != 0)
    def _():
        out_ref[...] += contrib


def _grouped_mlp(xs, block_g, W1, b1, W2, b2, cap, interpret=False):
    nb = cap // BLK_M
    nf = F // BLK_F
    grid_spec = pltpu.PrefetchScalarGridSpec(
        num_scalar_prefetch=1,
        grid=(nb, nf),
        in_specs=[
            pl.BlockSpec((BLK_M, D), lambda b, f, g: (b, 0)),
            pl.BlockSpec((1, D, BLK_F), lambda b, f, g: (g[b], 0, f)),
            pl.BlockSpec((1, 1, BLK_F), lambda b, f, g: (g[b], 0, f)),
            pl.BlockSpec((1, BLK_F, D), lambda b, f, g: (g[b], f, 0)),
            pl.BlockSpec((1, 1, D), lambda b, f, g: (g[b], 0, 0)),
        ],
        out_specs=pl.BlockSpec((BLK_M, D), lambda b, f, g: (b, 0)),
    )
    return pl.pallas_call(
        _mlp_body,
        grid_spec=grid_spec,
        out_shape=jax.ShapeDtypeStruct((cap, D), jnp.float32),
        interpret=interpret,
    )(block_g, xs, W1, b1.reshape(E, 1, F), W2, b2.reshape(E, 1, D))


@functools.partial(jax.jit, static_argnames=("interpret",))
def kernel(x, gate_w, W1, b1, W2, b2, interpret=False):
    orig_shape = x.shape
    xf = x.reshape(-1, D)
    n = xf.shape[0]
    nk = n * K

    scores = xf @ gate_w.T
    ew, ei = jax.lax.top_k(scores, K)
    w = jax.nn.softmax(ew, axis=-1)

    flat_e = ei.reshape(-1)
    order = jnp.argsort(flat_e, stable=True)
    sizes = jnp.bincount(flat_e, length=E)
    padded = ((sizes + BLK_M - 1) // BLK_M) * BLK_M
    pad_starts = jnp.cumsum(padded) - padded
    starts = jnp.cumsum(sizes) - sizes
    cap = nk + E * BLK_M
    nb = cap // BLK_M

    g_sorted = flat_e[order]
    pos_sorted = pad_starts[g_sorted] + (jnp.arange(nk, dtype=jnp.int32)
                                         - starts[g_sorted])
    src_tok = jnp.zeros((cap,), jnp.int32).at[pos_sorted].set(
        (order // K).astype(jnp.int32))
    pos = jnp.zeros((nk,), jnp.int32).at[order].set(pos_sorted.astype(jnp.int32))
    block_start = jnp.arange(nb, dtype=jnp.int32) * BLK_M
    block_g = ((block_start[:, None] >= pad_starts[None, :]).sum(-1) - 1
               ).astype(jnp.int32)

    xs = xf[src_tok]

    ys = _grouped_mlp(xs, block_g, W1, b1, W2, b2, cap, interpret=interpret)

    pos2 = pos.reshape(n, K)
    out = (ys[pos2] * w[:, :, None]).sum(axis=1)
    return out.reshape(orig_shape)


# speedup vs baseline: 2.5819x; 1.1609x over previous
import functools

import jax
import jax.numpy as jnp
from jax.experimental import pallas as pl
from jax.experimental.pallas import tpu as pltpu

E = 8
K = 2
D = 1024
F = 4096

BLK_M = 256
BLK_F = 1024


def _mlp_body(gmap, xs_ref, w1_ref, b1_ref, w2_ref, b2_ref, out_ref):
    x = xs_ref[...]
    h = jnp.dot(x, w1_ref[0], preferred_element_type=jnp.float32) + b1_ref[0]
    h = jax.nn.gelu(h).astype(jnp.bfloat16)
    out_ref[...] = (jnp.dot(h, w2_ref[0], preferred_element_type=jnp.float32)
                    + b2_ref[0])


def _grouped_mlp(xs, block_g, W1, b1, W2, b2, cap, interpret=False):
    nb = cap // BLK_M
    grid_spec = pltpu.PrefetchScalarGridSpec(
        num_scalar_prefetch=1,
        grid=(nb,),
        in_specs=[
            pl.BlockSpec((BLK_M, D), lambda b, g: (b, 0)),
            pl.BlockSpec((1, D, F), lambda b, g: (g[b], 0, 0)),
            pl.BlockSpec((1, 1, F), lambda b, g: (g[b], 0, 0)),
            pl.BlockSpec((1, F, D), lambda b, g: (g[b], 0, 0)),
            pl.BlockSpec((1, 1, D), lambda b, g: (g[b], 0, 0)),
        ],
        out_specs=pl.BlockSpec((BLK_M, D), lambda b, g: (b, 0)),
    )
    return pl.pallas_call(
        _mlp_body,
        grid_spec=grid_spec,
        out_shape=jax.ShapeDtypeStruct((cap, D), jnp.float32),
        interpret=interpret,
    )(block_g, xs.astype(jnp.bfloat16), W1.astype(jnp.bfloat16),
      b1.reshape(E, 1, F), W2.astype(jnp.bfloat16), b2.reshape(E, 1, D))


@functools.partial(jax.jit, static_argnames=("interpret",))
def kernel(x, gate_w, W1, b1, W2, b2, interpret=False):
    orig_shape = x.shape
    xf = x.reshape(-1, D)
    n = xf.shape[0]
    nk = n * K

    scores = xf @ gate_w.T
    ew, ei = jax.lax.top_k(scores, K)
    w = jax.nn.softmax(ew, axis=-1)

    flat_e = ei.reshape(-1)
    order = jnp.argsort(flat_e, stable=True)
    sizes = jnp.bincount(flat_e, length=E)
    padded = ((sizes + BLK_M - 1) // BLK_M) * BLK_M
    pad_starts = jnp.cumsum(padded) - padded
    starts = jnp.cumsum(sizes) - sizes
    cap = nk + E * BLK_M
    nb = cap // BLK_M

    g_sorted = flat_e[order]
    pos_sorted = pad_starts[g_sorted] + (jnp.arange(nk, dtype=jnp.int32)
                                         - starts[g_sorted])
    src_tok = jnp.zeros((cap,), jnp.int32).at[pos_sorted].set(
        (order // K).astype(jnp.int32))
    pos = jnp.zeros((nk,), jnp.int32).at[order].set(pos_sorted.astype(jnp.int32))
    block_start = jnp.arange(nb, dtype=jnp.int32) * BLK_M
    block_g = ((block_start[:, None] >= pad_starts[None, :]).sum(-1) - 1
               ).astype(jnp.int32)

    xs = xf[src_tok]

    ys = _grouped_mlp(xs, block_g, W1, b1, W2, b2, cap, interpret=interpret)

    pos2 = pos.reshape(n, K)
    out = (ys[pos2] * w[:, :, None]).sum(axis=1)
    return out.reshape(orig_shape)


# speedup vs baseline: 2.7804x; 1.0769x over previous
import functools

import jax
import jax.numpy as jnp
from jax.experimental import pallas as pl
from jax.experimental.pallas import tpu as pltpu

E = 8
K = 2
D = 1024
F = 4096

BLK_M = 256
BLK_F = 1024


def _mlp_body(gmap, xs_ref, w1_ref, b1_ref, w2_ref, b2_ref, out_ref):
    x = xs_ref[...]
    h = jnp.dot(x, w1_ref[0], preferred_element_type=jnp.float32) + b1_ref[0]
    h = jax.nn.gelu(h).astype(jnp.bfloat16)
    out_ref[...] = (jnp.dot(h, w2_ref[0], preferred_element_type=jnp.float32)
                    + b2_ref[0])


def _grouped_mlp(xs, block_g, W1, b1, W2, b2, cap, interpret=False):
    nb = cap // BLK_M
    grid_spec = pltpu.PrefetchScalarGridSpec(
        num_scalar_prefetch=1,
        grid=(nb,),
        in_specs=[
            pl.BlockSpec((BLK_M, D), lambda b, g: (b, 0)),
            pl.BlockSpec((1, D, F), lambda b, g: (g[b], 0, 0)),
            pl.BlockSpec((1, 1, F), lambda b, g: (g[b], 0, 0)),
            pl.BlockSpec((1, F, D), lambda b, g: (g[b], 0, 0)),
            pl.BlockSpec((1, 1, D), lambda b, g: (g[b], 0, 0)),
        ],
        out_specs=pl.BlockSpec((BLK_M, D), lambda b, g: (b, 0)),
    )
    return pl.pallas_call(
        _mlp_body,
        grid_spec=grid_spec,
        out_shape=jax.ShapeDtypeStruct((cap, D), jnp.float32),
        interpret=interpret,
    )(block_g, xs.astype(jnp.bfloat16), W1.astype(jnp.bfloat16),
      b1.reshape(E, 1, F), W2.astype(jnp.bfloat16), b2.reshape(E, 1, D))


@functools.partial(jax.jit, static_argnames=("interpret",))
def kernel(x, gate_w, W1, b1, W2, b2, interpret=False):
    orig_shape = x.shape
    xf = x.reshape(-1, D)
    n = xf.shape[0]
    nk = n * K

    scores = xf @ gate_w.T
    i1 = jnp.argmax(scores, axis=-1).astype(jnp.int32)
    m1 = jnp.max(scores, axis=-1)
    masked = jnp.where(jax.nn.one_hot(i1, E, dtype=jnp.bool_), -jnp.inf, scores)
    i2 = jnp.argmax(masked, axis=-1).astype(jnp.int32)
    m2 = jnp.max(masked, axis=-1)
    ei = jnp.stack([i1, i2], axis=-1)
    w = jax.nn.softmax(jnp.stack([m1, m2], axis=-1), axis=-1)

    flat_e = ei.reshape(-1)
    oh = (flat_e[:, None] == jnp.arange(E, dtype=jnp.int32)[None, :]
          ).astype(jnp.int32)
    csum = jnp.cumsum(oh, axis=0)
    sizes = csum[-1]
    rank = ((csum - oh) * oh).sum(-1)
    padded = ((sizes + BLK_M - 1) // BLK_M) * BLK_M
    pad_starts = jnp.cumsum(padded) - padded
    cap = nk + E * BLK_M
    nb = cap // BLK_M

    pos = (pad_starts[flat_e] + rank).astype(jnp.int32)
    src_tok = jnp.zeros((cap,), jnp.int32).at[pos].set(
        (jnp.arange(nk, dtype=jnp.int32) // K))
    block_start = jnp.arange(nb, dtype=jnp.int32) * BLK_M
    block_g = ((block_start[:, None] >= pad_starts[None, :]).sum(-1) - 1
               ).astype(jnp.int32)

    xs = xf[src_tok]

    ys = _grouped_mlp(xs, block_g, W1, b1, W2, b2, cap, interpret=interpret)

    pos2 = pos.reshape(n, K)
    out = (ys[pos2] * w[:, :, None]).sum(axis=1)
    return out.reshape(orig_shape)


# speedup vs baseline: 2.9915x; 1.0759x over previous
import functools

import jax
import jax.numpy as jnp
from jax import lax
from jax.experimental import pallas as pl
from jax.experimental.pallas import tpu as pltpu
from jax.experimental.pallas import tpu_sc as plsc

E = 8
K = 2
D = 1024
F = 4096

BLK_M = 256
BLK_F = 1024


def _mlp_body(gmap, xs_ref, w1_ref, b1_ref, w2_ref, b2_ref, wcol_ref, out_ref):
    x = xs_ref[...]
    h = jnp.dot(x, w1_ref[0], preferred_element_type=jnp.float32) + b1_ref[0]
    h = jax.nn.gelu(h).astype(jnp.bfloat16)
    out_ref[...] = ((jnp.dot(h, w2_ref[0], preferred_element_type=jnp.float32)
                     + b2_ref[0]) * wcol_ref[...])


def _grouped_mlp(xs, block_g, W1, b1, W2, b2, wcol, cap):
    nb = cap // BLK_M
    grid_spec = pltpu.PrefetchScalarGridSpec(
        num_scalar_prefetch=1,
        grid=(nb,),
        in_specs=[
            pl.BlockSpec((BLK_M, D), lambda b, g: (b, 0)),
            pl.BlockSpec((1, D, F), lambda b, g: (g[b], 0, 0)),
            pl.BlockSpec((1, 1, F), lambda b, g: (g[b], 0, 0)),
            pl.BlockSpec((1, F, D), lambda b, g: (g[b], 0, 0)),
            pl.BlockSpec((1, 1, D), lambda b, g: (g[b], 0, 0)),
            pl.BlockSpec((BLK_M, 1), lambda b, g: (b, 0)),
        ],
        out_specs=pl.BlockSpec((BLK_M, D), lambda b, g: (b, 0)),
    )
    return pl.pallas_call(
        _mlp_body,
        grid_spec=grid_spec,
        out_shape=jax.ShapeDtypeStruct((cap, D), jnp.float32),
    )(block_g, xs.astype(jnp.bfloat16), W1.astype(jnp.bfloat16),
      b1.reshape(E, 1, F), W2.astype(jnp.bfloat16), b2.reshape(E, 1, D),
      wcol)


def _sc_combine(ys, pos0, pos1, n):
    info = plsc.get_sparse_core_info()
    nw = info.num_cores * info.num_subcores
    tpw = n // nw
    T = 32
    nchunk = tpw // T
    mesh = plsc.VectorSubcoreMesh(core_axis_name="c", subcore_axis_name="s")

    @functools.partial(
        pl.kernel, mesh=mesh,
        out_type=jax.ShapeDtypeStruct((n, D), jnp.float32),
        scratch_types=[
            pltpu.VMEM((T,), jnp.int32),
            pltpu.VMEM((T,), jnp.int32),
            pltpu.VMEM((T, D), jnp.float32),
            pltpu.VMEM((T, D), jnp.float32),
            pltpu.VMEM((T, D), jnp.float32),
            pltpu.SemaphoreType.DMA,
        ],
    )
    def k(ys_hbm, pos0_hbm, pos1_hbm, out_hbm,
          idx0_v, idx1_v, rows0_v, rows1_v, out_v, sem):
        wid = lax.axis_index("s") * info.num_cores + lax.axis_index("c")
        base = wid * tpw
        for c in range(nchunk):
            off = base + c * T
            pltpu.sync_copy(pos0_hbm.at[pl.ds(off, T)], idx0_v)
            pltpu.sync_copy(pos1_hbm.at[pl.ds(off, T)], idx1_v)
            cp0 = pltpu.async_copy(ys_hbm.at[idx0_v], rows0_v, sem)
            cp1 = pltpu.async_copy(ys_hbm.at[idx1_v], rows1_v, sem)
            cp0.wait()
            cp1.wait()

            def body(t, _):
                for j in range(D // 16):
                    sl = pl.ds(j * 16, 16)
                    out_v[t, sl] = rows0_v[t, sl] + rows1_v[t, sl]
                return _

            lax.fori_loop(0, T, body, 0)
            pltpu.sync_copy(out_v, out_hbm.at[pl.ds(off, T)])

    return k(ys, pos0, pos1)


@jax.jit
def kernel(x, gate_w, W1, b1, W2, b2):
    orig_shape = x.shape
    xf = x.reshape(-1, D)
    n = xf.shape[0]
    nk = n * K

    scores = xf @ gate_w.T
    i1 = jnp.argmax(scores, axis=-1).astype(jnp.int32)
    m1 = jnp.max(scores, axis=-1)
    masked = jnp.where(jax.nn.one_hot(i1, E, dtype=jnp.bool_), -jnp.inf, scores)
    i2 = jnp.argmax(masked, axis=-1).astype(jnp.int32)
    m2 = jnp.max(masked, axis=-1)
    ei = jnp.stack([i1, i2], axis=-1)
    w = jax.nn.softmax(jnp.stack([m1, m2], axis=-1), axis=-1)

    flat_e = ei.reshape(-1)
    oh = (flat_e[:, None] == jnp.arange(E, dtype=jnp.int32)[None, :]
          ).astype(jnp.int32)
    csum = jnp.cumsum(oh, axis=0)
    sizes = csum[-1]
    rank = ((csum - oh) * oh).sum(-1)
    padded = ((sizes + BLK_M - 1) // BLK_M) * BLK_M
    pad_starts = jnp.cumsum(padded) - padded
    cap = nk + E * BLK_M
    nb = cap // BLK_M

    pos = (pad_starts[flat_e] + rank).astype(jnp.int32)
    src_tok = jnp.zeros((cap,), jnp.int32).at[pos].set(
        (jnp.arange(nk, dtype=jnp.int32) // K))
    block_start = jnp.arange(nb, dtype=jnp.int32) * BLK_M
    block_g = ((block_start[:, None] >= pad_starts[None, :]).sum(-1) - 1
               ).astype(jnp.int32)

    wcol = jnp.zeros((cap,), jnp.float32).at[pos].set(w.reshape(-1))

    xs = xf[src_tok]

    ys = _grouped_mlp(xs, block_g, W1, b1, W2, b2, wcol.reshape(cap, 1), cap)

    pos2 = pos.reshape(n, K)
    out = _sc_combine(ys, pos2[:, 0], pos2[:, 1], n)
    return out.reshape(orig_shape)
